# Initial kernel scaffold; baseline (speedup 1.0000x reference)
import jax
import jax.numpy as jnp
from jax.experimental import pallas as pl


def kernel(user_table, age_table, device_table, gender_table, article_table, type_table, topic_table, cat_table, subcat_table, sentiment_table, multimodal_table, content_W, time_W, time_b, din_W1, din_b1, din_W2, din_b2, pop_W1, pop_b1, pop_W2, pop_b2, gate_W1, gate_b1, gate_W2, gate_b2, mlp_W1, mlp_b1, mlp_W2, mlp_b2, mlp_W3, mlp_b3, mlp_W4, mlp_b4, user_id, age, gender, device, target_id, history_ids, history_category, target_type, target_topics, target_category, target_subcat, target_sentiment_label, target_published_ts, imp_time):
    raise NotImplementedError("write your pallas kernel here")


# speedup vs baseline: 1.1323x; 1.1323x over previous
import jax
import jax.numpy as jnp
from jax import lax
from jax.experimental import pallas as pl
from jax.experimental.pallas import tpu as pltpu
from jax.experimental.pallas import tpu_sc as plsc

B = 1024
L = 50
T = 5
D = 64
RAW = 128
HN = B * L

NC, NS = 2, 16
NW = NC * NS
BW = B // NW
HW = HN // NW
HC = 80
NCHUNK = HW // HC
TW = B * T // NW
TCH = 80
NTCH = TW // TCH


def _sc_body(user_t, age_t, gender_t, device_t, article_t, type_t, cat_t,
             subcat_t, sent_t, topic_t, mm_t,
             user_id, age, gender, device, target_id, target_type,
             target_cat, target_subcat, target_sent,
             topics_i, hist_i, histcat_i,
             user_o, age_o, gender_o, device_o, artT_o, type_o, catT_o,
             subcat_o, sent_o, mmT_o, topics_o, artH_o, catH_o, mmH_o,
             i0, i1, i2, i3, i4, i5, i6, i7, i8, i9,
             r0, r1, r2, r3, r4, r5, r6, r7, r8, r9,
             hidx, hcidx, ha, hc, hm, gsem, wsem):
    wid = lax.axis_index("s") * NC + lax.axis_index("c")
    b0 = wid * BW

    small = [
        (user_t, user_id, user_o, i0, r0),
        (age_t, age, age_o, i1, r1),
        (gender_t, gender, gender_o, i2, r2),
        (device_t, device, device_o, i3, r3),
        (article_t, target_id, artT_o, i4, r4),
        (type_t, target_type, type_o, i5, r5),
        (cat_t, target_cat, catT_o, i6, r6),
        (subcat_t, target_subcat, subcat_o, i7, r7),
        (sent_t, target_sent, sent_o, i8, r8),
        (mm_t, target_id, mmT_o, i9, r9),
    ]
    ds = [pltpu.async_copy(idx.at[pl.ds(b0, BW)], ib, gsem)
          for (_, idx, _, ib, _) in small]
    for d in ds:
        d.wait()
    ds = [pltpu.async_copy(tab.at[ib], rb, gsem)
          for (tab, _, _, ib, rb) in small]
    for d in ds:
        d.wait()
    ds = [pltpu.async_copy(rb, out.at[pl.ds(b0, BW)], wsem)
          for (_, _, out, _, rb) in small]
    for d in ds:
        d.wait()

    t0 = wid * TW
    for c in range(NTCH):
        off = t0 + c * TCH
        pltpu.sync_copy(topics_i.at[pl.ds(off, TCH)], hidx)
        pltpu.async_copy(topic_t.at[hidx], ha, gsem).wait()
        pltpu.sync_copy(ha, topics_o.at[pl.ds(off, TCH)])

    h0 = wid * HW
    for c in range(NCHUNK):
        off = h0 + c * HC
        d1 = pltpu.async_copy(hist_i.at[pl.ds(off, HC)], hidx, gsem)
        d2 = pltpu.async_copy(histcat_i.at[pl.ds(off, HC)], hcidx, gsem)
        d1.wait()
        d2.wait()
        g1 = pltpu.async_copy(article_t.at[hidx], ha, gsem)
        g2 = pltpu.async_copy(cat_t.at[hcidx], hc, gsem)
        g3 = pltpu.async_copy(mm_t.at[hidx], hm, gsem)
        g1.wait()
        g2.wait()
        g3.wait()
        w1 = pltpu.async_copy(ha, artH_o.at[pl.ds(off, HC)], wsem)
        w2 = pltpu.async_copy(hc, catH_o.at[pl.ds(off, HC)], wsem)
        w3 = pltpu.async_copy(hm, mmH_o.at[pl.ds(off, HC)], wsem)
        w1.wait()
        w2.wait()
        w3.wait()


def _make_sc_gather():
    f32 = jnp.float32
    i32 = jnp.int32
    mesh = plsc.VectorSubcoreMesh(core_axis_name="c", subcore_axis_name="s",
                                  num_cores=NC, num_subcores=NS)
    out_type = (
        [jax.ShapeDtypeStruct((B, D), f32) for _ in range(9)]
        + [jax.ShapeDtypeStruct((B, RAW), f32),
           jax.ShapeDtypeStruct((B * T, D), f32),
           jax.ShapeDtypeStruct((HN, D), f32),
           jax.ShapeDtypeStruct((HN, D), f32),
           jax.ShapeDtypeStruct((HN, RAW), f32)]
    )
    scratch_types = (
        [pltpu.VMEM((BW,), i32) for _ in range(10)]
        + [pltpu.VMEM((BW, D), f32) for _ in range(9)]
        + [pltpu.VMEM((BW, RAW), f32),
           pltpu.VMEM((HC,), i32), pltpu.VMEM((HC,), i32),
           pltpu.VMEM((HC, D), f32), pltpu.VMEM((HC, D), f32),
           pltpu.VMEM((HC, RAW), f32),
           pltpu.SemaphoreType.DMA, pltpu.SemaphoreType.DMA]
    )
    return pl.kernel(_sc_body, out_type=out_type, mesh=mesh,
                     scratch_types=scratch_types,
                     compiler_params=pltpu.CompilerParams(
                         use_tc_tiling_on_sc=False))


Bb = 128
GRID = B // Bb


def _tc_body(user_e, age_e, gender_e, device_e, artT, typeE, catT, subcatE,
             sentE, topicsE, mmT, artH, catH, mmH, hist_ids, ts,
             content_W, W1qd, W1kd, W1m, b1, W2row, b2,
             timeW, timeB, popW1, popB1, popW2row, popB2,
             gateW1, gateB1, gateW2row, gateB2,
             mW1, mB1, mW2, mB2, mW3, mB3, mW4row, mB4,
             y_o, din_o, pop_o, alpha_o):
    f32 = jnp.float32

    def dot(a, b):
        return lax.dot_general(a, b, (((1,), (0,)), ((), ())),
                               preferred_element_type=f32)

    def sigmoid(z):
        return 1.0 / (1.0 + jnp.exp(-z))

    cw = content_W[...]
    tc = dot(mmT[...], cw)
    hcon = dot(mmH[...], cw)
    q192 = jnp.concatenate([artT[...], catT[...], tc], axis=-1)
    k192 = jnp.concatenate([artH[...], catH[...], hcon], axis=-1)
    q_term = dot(q192, W1qd[...]) + b1[...]
    k_term = dot(k192, W1kd[...])
    q3 = jnp.broadcast_to(q192.reshape(Bb, 1, 192), (Bb, L, 192))
    qk = q3.reshape(Bb * L, 192) * k192
    m_term = dot(qk, W1m[...])
    qt = jnp.broadcast_to(q_term.reshape(Bb, 1, D), (Bb, L, D))
    h = jnp.maximum(k_term + m_term + qt.reshape(Bb * L, D), 0.0)
    s = jnp.sum(h.reshape(Bb, L, D) * W2row[...].reshape(1, 1, D), axis=2)
    s = s + b2[...]
    mask = hist_ids[...] != 0
    s = jnp.where(mask, s, -1e9)
    smax = jnp.max(s, axis=1, keepdims=True)
    e = jnp.exp(s - smax)
    w = e / jnp.sum(e, axis=1, keepdims=True)
    hist_att = jnp.sum(w.reshape(Bb, L, 1) * k192.reshape(Bb, L, 192), axis=1)

    te = topicsE[...]
    topics_sum = (te[:, 0:64] + te[:, 64:128] + te[:, 128:192]
                  + te[:, 192:256] + te[:, 256:320])
    user_c = jnp.concatenate([user_e[...], age_e[...], gender_e[...],
                              device_e[...]], axis=-1)
    target_c = jnp.concatenate([artT[...], typeE[...], catT[...],
                                subcatE[...], sentE[...], topics_sum],
                               axis=-1)
    x = jnp.concatenate([user_c, target_c, hist_att], axis=-1)
    x = jnp.maximum(dot(x, mW1[...]) + mB1[...], 0.0)
    x = jnp.maximum(dot(x, mW2[...]) + mB2[...], 0.0)
    x = jnp.maximum(dot(x, mW3[...]) + mB3[...], 0.0)
    din = sigmoid(jnp.sum(x * mW4row[...], axis=1, keepdims=True) + mB4[...])

    ts_emb = ts[...] * timeW[...] + timeB[...]
    pop_in = jnp.concatenate([ts_emb, tc], axis=-1)
    pop_h = jnp.maximum(dot(pop_in, popW1[...]) + popB1[...], 0.0)
    pop = sigmoid(jnp.sum(pop_h * popW2row[...], axis=1, keepdims=True)
                  + popB2[...])

    gate_in = jnp.concatenate([user_c, ts_emb, tc], axis=-1)
    gate_h = jnp.maximum(dot(gate_in, gateW1[...]) + gateB1[...], 0.0)
    alpha = sigmoid(jnp.sum(gate_h * gateW2row[...], axis=1, keepdims=True)
                    + gateB2[...])

    y_o[...] = alpha * din + (1.0 - alpha) * pop
    din_o[...] = din
    pop_o[...] = pop
    alpha_o[...] = alpha


def _tc_call(user_e, age_e, gender_e, device_e, artT, typeE, catT, subcatE,
             sentE, topicsE, mmT, artH, catH, mmH, hist_ids, ts, weights,
             interpret=False):
    f32 = jnp.float32

    def blk(d):
        return pl.BlockSpec((Bb, d), lambda i: (i, 0))

    def hblk(d):
        return pl.BlockSpec((Bb * L, d), lambda i: (i, 0))

    def const(shape):
        return pl.BlockSpec(shape, lambda i: (0,) * len(shape))

    in_specs = ([blk(D)] * 9 + [blk(T * D), blk(RAW), hblk(D), hblk(D),
                hblk(RAW), blk(L), blk(1)]
                + [const(wt.shape) for wt in weights])
    out_specs = [blk(1)] * 4
    out_shape = [jax.ShapeDtypeStruct((B, 1), f32)] * 4
    fn = pl.pallas_call(
        _tc_body,
        grid=(GRID,),
        in_specs=in_specs,
        out_specs=out_specs,
        out_shape=out_shape,
        interpret=interpret,
    )
    return fn(user_e, age_e, gender_e, device_e, artT, typeE, catT, subcatE,
              sentE, topicsE, mmT, artH, catH, mmH, hist_ids, ts, *weights)


def _prep_weights(content_W, time_W, time_b, din_W1, din_b1, din_W2, din_b2,
                  pop_W1, pop_b1, pop_W2, pop_b2, gate_W1, gate_b1, gate_W2,
                  gate_b2, mlp_W1, mlp_b1, mlp_W2, mlp_b2, mlp_W3, mlp_b3,
                  mlp_W4, mlp_b4):
    c = 1.0 / jnp.sqrt(jnp.float32(1.0 + 1e-5))
    W1q, W1k, W1d, W1m = (din_W1[0:192], din_W1[192:384],
                          din_W1[384:576], din_W1[576:768])
    return [
        content_W,
        W1q + W1d, W1k - W1d, W1m, din_b1.reshape(1, D),
        din_W2.reshape(1, D), din_b2.reshape(1, 1),
        time_W.reshape(1, D), time_b.reshape(1, D),
        pop_W1, pop_b1.reshape(1, D), pop_W2.reshape(1, D),
        pop_b2.reshape(1, 1),
        gate_W1, gate_b1.reshape(1, D), gate_W2.reshape(1, D),
        gate_b2.reshape(1, 1),
        mlp_W1 * c, mlp_b1.reshape(1, 512) * c,
        mlp_W2 * c, mlp_b2.reshape(1, 256) * c,
        mlp_W3 * c, mlp_b3.reshape(1, 128) * c,
        mlp_W4.reshape(1, 128), mlp_b4.reshape(1, 1),
    ]


def kernel(user_table, age_table, device_table, gender_table, article_table,
           type_table, topic_table, cat_table, subcat_table, sentiment_table,
           multimodal_table, content_W, time_W, time_b, din_W1, din_b1,
           din_W2, din_b2, pop_W1, pop_b1, pop_W2, pop_b2, gate_W1, gate_b1,
           gate_W2, gate_b2, mlp_W1, mlp_b1, mlp_W2, mlp_b2, mlp_W3, mlp_b3,
           mlp_W4, mlp_b4, user_id, age, gender, device, target_id,
           history_ids, history_category, target_type, target_topics,
           target_category, target_subcat, target_sentiment_label,
           target_published_ts, imp_time):
    i32 = jnp.int32
    user_id = user_id.astype(i32)
    age = age.astype(i32)
    gender = gender.astype(i32)
    device = device.astype(i32)
    target_id = target_id.astype(i32)
    target_type = target_type.astype(i32)
    target_category = target_category.astype(i32)
    target_subcat = target_subcat.astype(i32)
    target_sent = target_sentiment_label.astype(i32)
    hist_ids2d = history_ids.astype(i32)
    topics_flat = target_topics.astype(i32).reshape(B * T)
    hist_flat = hist_ids2d.reshape(HN)
    histcat_flat = history_category.astype(i32).reshape(HN)

    gather = _make_sc_gather()
    (user_e, age_e, gender_e, device_e, artT, typeE, catT, subcatE, sentE,
     mmT, topicsE, artH, catH, mmH) = gather(
        user_table, age_table, gender_table, device_table, article_table,
        type_table, cat_table, subcat_table, sentiment_table, topic_table,
        multimodal_table,
        user_id, age, gender, device, target_id, target_type,
        target_category, target_subcat, target_sent,
        topics_flat, hist_flat, histcat_flat)

    weights = _prep_weights(content_W, time_W, time_b, din_W1, din_b1,
                            din_W2, din_b2, pop_W1, pop_b1, pop_W2, pop_b2,
                            gate_W1, gate_b1, gate_W2, gate_b2, mlp_W1,
                            mlp_b1, mlp_W2, mlp_b2, mlp_W3, mlp_b3, mlp_W4,
                            mlp_b4)
    y, din, pop, alpha = _tc_call(
        user_e, age_e, gender_e, device_e, artT, typeE, catT, subcatE, sentE,
        topicsE.reshape(B, T * D), mmT, artH, catH, mmH, hist_ids2d,
        target_published_ts, weights)
    return (y, din, pop, alpha)
